# baseline (speedup 1.0000x reference)
import functools

import jax
import jax.numpy as jnp
from jax import lax
from jax.experimental import pallas as pl
from jax.experimental.pallas import tpu as pltpu
from jax.experimental.pallas import tpu_sc as plsc

N = 10000
E = 320000
DE = 16
D = 128
DH = 64

_TILES = 16
_EK = 80
_ROWS_PER_TILE = N // _TILES
_STAGE = 125


def _ew_body(ef_ref, wrow_ref, bet_ref, out_ref):
    blk = ef_ref[...]
    wrow = wrow_ref[...]
    wvec = jnp.concatenate([wrow] * 8, axis=1)
    ii = jax.lax.broadcasted_iota(jnp.int32, (D, 8), 0)
    jj = jax.lax.broadcasted_iota(jnp.int32, (D, 8), 1)
    P = (ii // DE == jj).astype(jnp.float32)
    y = jnp.dot(blk * wvec, P, preferred_element_type=jnp.float32)
    out_ref[...] = jax.nn.softplus(y + bet_ref[0])


def _edge_weights(edge_feats, Wet, bet):
    BE = 2000
    ef8 = edge_feats.reshape(E // 8, D)
    out = pl.pallas_call(
        _ew_body,
        grid=(E // 8 // BE,),
        in_specs=[
            pl.BlockSpec((BE, D), lambda i: (i, 0)),
            pl.BlockSpec((1, DE), lambda i: (0, 0)),
            pl.BlockSpec(memory_space=pltpu.SMEM),
        ],
        out_specs=pl.BlockSpec((BE, 8), lambda i: (i, 0)),
        out_shape=jax.ShapeDtypeStruct((E // 8, 8), jnp.float32),
    )(ef8, Wet.reshape(1, DE), bet)
    return out.reshape(E)


_DCH = E // 2 // _TILES // _EK


def _degree_sc(col4, ew4):
    mesh = plsc.VectorSubcoreMesh(core_axis_name="c", subcore_axis_name="s")

    @functools.partial(
        pl.kernel,
        out_type=jax.ShapeDtypeStruct((2 * N,), jnp.float32),
        mesh=mesh,
        compiler_params=pltpu.CompilerParams(
            needs_layout_passes=False, use_tc_tiling_on_sc=False),
        scratch_types=dict(
            deg_sp=pltpu.VMEM_SHARED((N,), jnp.float32),
            col_res=pltpu.VMEM((_DCH, _EK), jnp.int32),
            ew_res=pltpu.VMEM((_DCH, _EK), jnp.float32),
            buf_v=pltpu.VMEM((_EK,), jnp.float32),
        ),
    )
    def run(col_h, ew_h, out_h, *, deg_sp, col_res, ew_res, buf_v):
        c = lax.axis_index("c")
        s = lax.axis_index("s")

        pltpu.sync_copy(col_h.at[c, s], col_res)
        pltpu.sync_copy(ew_h.at[c, s], ew_res)

        for q in range(_EK // 16):
            buf_v[pl.ds(q * 16, 16)] = jnp.zeros((16,), jnp.float32)

        n_node_chunks = N // _EK
        n_iter = (n_node_chunks + _TILES - 1) // _TILES

        def zero_body(m, _):
            j = s + m * _TILES

            @pl.when(j < n_node_chunks)
            def _():
                off = pl.multiple_of(j * _EK, 8)
                pltpu.sync_copy(buf_v, deg_sp.at[pl.ds(off, _EK)])
            return _
        lax.fori_loop(0, n_iter, zero_body, None, unroll=False)

        plsc.subcore_barrier()

        def edge_body(j, _):
            pltpu.sync_copy(ew_res.at[j], deg_sp.at[col_res.at[j]], add=True)
            return _
        lax.fori_loop(0, _DCH, edge_body, None, unroll=False)

        plsc.subcore_barrier()

        def out_body(m, _):
            j = s + m * _TILES

            @pl.when(j < n_node_chunks)
            def _():
                off = pl.multiple_of(j * _EK, 8)
                pltpu.sync_copy(deg_sp.at[pl.ds(off, _EK)], buf_v)
                out_off = pl.multiple_of(c * N + off, 8)
                pltpu.sync_copy(buf_v, out_h.at[pl.ds(out_off, _EK)])
            return _
        lax.fori_loop(0, n_iter, out_body, None, unroll=False)

    return run(col4, ew4)


_NCH = E // _TILES // _EK
_SUP = 50
_NBUF = 5
_NST = 80


def _spmm_sc(row3, col3, ew3, gA, gB):
    mesh = plsc.VectorSubcoreMesh(core_axis_name="c", subcore_axis_name="s")

    @functools.partial(
        pl.kernel,
        out_type=[jax.ShapeDtypeStruct((N, DH), jnp.float32),
                  jax.ShapeDtypeStruct((N, DH), jnp.float32)],
        mesh=mesh,
        compiler_params=pltpu.CompilerParams(
            needs_layout_passes=False, use_tc_tiling_on_sc=False),
        scratch_types=dict(
            g_sp=pltpu.VMEM_SHARED((N, DH), jnp.float32),
            acc_sp=pltpu.VMEM_SHARED((N, DH), jnp.float32),
            ridx_res=pltpu.VMEM((_SUP, _EK), jnp.int32),
            cidx_res=pltpu.VMEM((_SUP, _EK), jnp.int32),
            ew_res=pltpu.VMEM((_SUP, _EK + 16), jnp.float32),
            rows_bufs=[pltpu.VMEM((_EK, DH), jnp.float32)
                       for _ in range(_NBUF)],
            stage_v=pltpu.VMEM((_NST, DH), jnp.float32),
            g_sems=[pltpu.SemaphoreType.DMA for _ in range(_NBUF)],
            s_sems=[pltpu.SemaphoreType.DMA for _ in range(_NBUF)],
        ),
    )
    def run(row_h, col_h, ew_h, ga_h, gb_h, sa_h, sb_h, *,
            g_sp, acc_sp, ridx_res, cidx_res, ew_res, rows_bufs,
            stage_v, g_sems, s_sems):
        c = lax.axis_index("c")
        s = lax.axis_index("s")
        n_node_chunks = N // _NST
        n_iter = (n_node_chunks + _TILES - 1) // _TILES

        def stage_in(g_h):
            def body(m, _):
                j = s + m * _TILES

                @pl.when(j < n_node_chunks)
                def _():
                    sl = pl.ds(pl.multiple_of(j * _NST, 8), _NST)
                    pltpu.sync_copy(g_h.at[sl, :], stage_v)
                    pltpu.sync_copy(stage_v, g_sp.at[sl, :])
                    pltpu.sync_copy(stage_v, acc_sp.at[sl, :])
                return _
            lax.fori_loop(0, n_iter, body, None, unroll=False)

        @pl.when(c == 0)
        def _():
            stage_in(ga_h)

        @pl.when(c == 1)
        def _():
            stage_in(gb_h)

        plsc.subcore_barrier()

        def scale(rows_ref, j):
            for e in range(_EK):
                bc = plsc.load_gather(
                    ew_res,
                    [jnp.full((16,), j, jnp.int32),
                     jnp.full((16,), e + 16, jnp.int32)])
                for q in range(DH // 16):
                    sl = pl.ds(q * 16, 16)
                    rows_ref[e, sl] = rows_ref[e, sl] * bc

        def gather_start(j, rows_ref, sem):
            return pltpu.async_copy(g_sp.at[ridx_res.at[j]], rows_ref, sem)

        def gather_wait(j, rows_ref, sem):
            pltpu.make_async_copy(
                g_sp.at[ridx_res.at[j]], rows_ref, sem).wait()

        def scatter_start(j, rows_ref, sem):
            return pltpu.async_copy(
                rows_ref, acc_sp.at[cidx_res.at[j]], sem, add=True)

        def scatter_wait(j, rows_ref, sem):
            pltpu.make_async_copy(
                rows_ref, acc_sp.at[cidx_res.at[j]], sem).wait()

        n_steps = _SUP // _NBUF
        last_t = n_steps - 1

        def super_body(u, _):
            pltpu.sync_copy(row_h.at[s, pl.ds(u * _SUP, _SUP)], ridx_res)
            pltpu.sync_copy(col_h.at[s, pl.ds(u * _SUP, _SUP)], cidx_res)
            pltpu.sync_copy(ew_h.at[s, pl.ds(u * _SUP, _SUP)],
                            ew_res.at[:, pl.ds(16, _EK)])
            for k in range(3):
                gather_start(k, rows_bufs[k], g_sems[k])

            def edge_body(t, _):
                for k in range(_NBUF):
                    j = t * _NBUF + k
                    buf = rows_bufs[k]
                    gather_wait(j, buf, g_sems[k])
                    scale(buf, j)
                    scatter_start(j, buf, s_sems[k])
                    nk = (k + 3) % _NBUF
                    if k >= 2:
                        scatter_wait(j - 2, rows_bufs[nk], s_sems[nk])

                        @pl.when(t < last_t)
                        def _():
                            gather_start(j + 3, rows_bufs[nk], g_sems[nk])
                    else:
                        @pl.when(t > 0)
                        def _():
                            scatter_wait(j - 2, rows_bufs[nk], s_sems[nk])
                        gather_start(j + 3, rows_bufs[nk], g_sems[nk])
                return _
            lax.fori_loop(0, n_steps, edge_body, None, unroll=False)
            scatter_wait(_SUP - 2, rows_bufs[(_SUP - 2) % _NBUF],
                         s_sems[(_SUP - 2) % _NBUF])
            scatter_wait(_SUP - 1, rows_bufs[(_SUP - 1) % _NBUF],
                         s_sems[(_SUP - 1) % _NBUF])
            return _
        lax.fori_loop(0, _NCH // _SUP, super_body, None, unroll=False)

        plsc.subcore_barrier()

        def stage_out(s_h):
            def body(m, _):
                j = s + m * _TILES

                @pl.when(j < n_node_chunks)
                def _():
                    sl = pl.ds(pl.multiple_of(j * _NST, 8), _NST)
                    pltpu.sync_copy(acc_sp.at[sl, :], stage_v)
                    pltpu.sync_copy(stage_v, s_h.at[sl, :])
                return _
            lax.fori_loop(0, n_iter, body, None, unroll=False)

        @pl.when(c == 0)
        def _():
            stage_out(sa_h)

        @pl.when(c == 1)
        def _():
            stage_out(sb_h)

    return run(row3, col3, ew3, gA, gB)


def _dg_body(p0_ref, p1_ref, x_ref, wa_ref, wb_ref, d_ref, ga_ref, gb_ref):
    d = jax.lax.rsqrt(p0_ref[...] + p1_ref[...] + 1.0)
    x = x_ref[...]
    ga_ref[...] = d * jnp.dot(x, wa_ref[...], preferred_element_type=jnp.float32)
    gb_ref[...] = d * jnp.dot(x, wb_ref[...], preferred_element_type=jnp.float32)
    d_ref[...] = d


def _d_and_g1(deg_parts, node_feats, W1):
    BN = 1000
    return pl.pallas_call(
        _dg_body,
        grid=(N // BN,),
        in_specs=[
            pl.BlockSpec((BN, 1), lambda i: (i, 0)),
            pl.BlockSpec((BN, 1), lambda i: (i, 0)),
            pl.BlockSpec((BN, D), lambda i: (i, 0)),
            pl.BlockSpec((D, DH), lambda i: (0, 0)),
            pl.BlockSpec((D, DH), lambda i: (0, 0)),
        ],
        out_specs=[
            pl.BlockSpec((BN, 1), lambda i: (i, 0)),
            pl.BlockSpec((BN, DH), lambda i: (i, 0)),
            pl.BlockSpec((BN, DH), lambda i: (i, 0)),
        ],
        out_shape=[
            jax.ShapeDtypeStruct((N, 1), jnp.float32),
            jax.ShapeDtypeStruct((N, DH), jnp.float32),
            jax.ShapeDtypeStruct((N, DH), jnp.float32),
        ],
    )(deg_parts[:N].reshape(N, 1), deg_parts[N:].reshape(N, 1),
      node_feats, W1[:, :DH], W1[:, DH:])


def _mid_body(sa_ref, sb_ref, d_ref, b1_ref, wa_ref, wb_ref, oa_ref, ob_ref):
    d = d_ref[...]
    b1 = b1_ref[...]
    xa = jnp.tanh(d * sa_ref[...] + b1[:, :DH])
    xb = jnp.tanh(d * sb_ref[...] + b1[:, DH:])
    x1 = jnp.concatenate([xa, xb], axis=1)
    oa_ref[...] = d * jnp.dot(x1, wa_ref[...], preferred_element_type=jnp.float32)
    ob_ref[...] = d * jnp.dot(x1, wb_ref[...], preferred_element_type=jnp.float32)


def _mid(SA, SB, dcol, b1, W2):
    BN = 1000
    return pl.pallas_call(
        _mid_body,
        grid=(N // BN,),
        in_specs=[
            pl.BlockSpec((BN, DH), lambda i: (i, 0)),
            pl.BlockSpec((BN, DH), lambda i: (i, 0)),
            pl.BlockSpec((BN, 1), lambda i: (i, 0)),
            pl.BlockSpec((1, D), lambda i: (0, 0)),
            pl.BlockSpec((D, DH), lambda i: (0, 0)),
            pl.BlockSpec((D, DH), lambda i: (0, 0)),
        ],
        out_specs=[
            pl.BlockSpec((BN, DH), lambda i: (i, 0)),
            pl.BlockSpec((BN, DH), lambda i: (i, 0)),
        ],
        out_shape=[
            jax.ShapeDtypeStruct((N, DH), jnp.float32),
            jax.ShapeDtypeStruct((N, DH), jnp.float32),
        ],
    )(SA, SB, dcol, b1.reshape(1, D), W2[:, :DH], W2[:, DH:])


def _fin_body(sa_ref, sb_ref, d_ref, b2_ref, x_ref, o_ref):
    d = d_ref[...]
    ya = d * sa_ref[...]
    yb = d * sb_ref[...]
    o_ref[...] = jnp.concatenate([ya, yb], axis=1) + b2_ref[...] + x_ref[...]


def _final(SA, SB, dcol, b2, node_feats):
    BN = 1000
    return pl.pallas_call(
        _fin_body,
        grid=(N // BN,),
        in_specs=[
            pl.BlockSpec((BN, DH), lambda i: (i, 0)),
            pl.BlockSpec((BN, DH), lambda i: (i, 0)),
            pl.BlockSpec((BN, 1), lambda i: (i, 0)),
            pl.BlockSpec((1, D), lambda i: (0, 0)),
            pl.BlockSpec((BN, D), lambda i: (i, 0)),
        ],
        out_specs=pl.BlockSpec((BN, D), lambda i: (i, 0)),
        out_shape=jax.ShapeDtypeStruct((N, D), jnp.float32),
    )(SA, SB, dcol, b2.reshape(1, D), node_feats)


def kernel(edge_index, edge_feats, node_feats, Wet, bet, W1, b1, W2, b2):
    row = edge_index[0]
    col = edge_index[1]
    ew = _edge_weights(edge_feats, Wet, bet)
    deg_parts = _degree_sc(col.reshape(2, _TILES, _DCH, _EK),
                           ew.reshape(2, _TILES, _DCH, _EK))
    dcol, g1A, g1B = _d_and_g1(deg_parts, node_feats, W1)
    row3 = row.reshape(_TILES, _NCH, _EK)
    col3 = col.reshape(_TILES, _NCH, _EK)
    ew3 = ew.reshape(_TILES, _NCH, _EK)
    S1A, S1B = _spmm_sc(row3, col3, ew3, g1A, g1B)
    g2A, g2B = _mid(S1A, S1B, dcol, b1, W2)
    S2A, S2B = _spmm_sc(row3, col3, ew3, g2A, g2B)
    return _final(S2A, S2B, dcol, b2, node_feats)


# speedup vs baseline: 1.9631x; 1.9631x over previous
import functools

import jax
import jax.numpy as jnp
from jax import lax
from jax.experimental import pallas as pl
from jax.experimental.pallas import tpu as pltpu
from jax.experimental.pallas import tpu_sc as plsc

N = 10000
E = 320000
DE = 16
D = 128
DH = 64

_TILES = 16
_EK = 80
_ROWS_PER_TILE = N // _TILES
_STAGE = 125


def _ew_body(ef_ref, wrow_ref, bet_ref, out_ref):
    blk = ef_ref[...]
    wrow = wrow_ref[...]
    wvec = jnp.concatenate([wrow] * 8, axis=1)
    ii = jax.lax.broadcasted_iota(jnp.int32, (D, 8), 0)
    jj = jax.lax.broadcasted_iota(jnp.int32, (D, 8), 1)
    P = (ii // DE == jj).astype(jnp.float32)
    y = jnp.dot(blk * wvec, P, preferred_element_type=jnp.float32)
    out_ref[...] = jax.nn.softplus(y + bet_ref[0])


def _edge_weights(edge_feats, Wet, bet):
    BE = 2000
    ef8 = edge_feats.reshape(E // 8, D)
    out = pl.pallas_call(
        _ew_body,
        grid=(E // 8 // BE,),
        in_specs=[
            pl.BlockSpec((BE, D), lambda i: (i, 0)),
            pl.BlockSpec((1, DE), lambda i: (0, 0)),
            pl.BlockSpec(memory_space=pltpu.SMEM),
        ],
        out_specs=pl.BlockSpec((BE, 8), lambda i: (i, 0)),
        out_shape=jax.ShapeDtypeStruct((E // 8, 8), jnp.float32),
    )(ef8, Wet.reshape(1, DE), bet)
    return out.reshape(E)


_DCH = E // 2 // _TILES // _EK


def _degree_sc(col4, ew4):
    mesh = plsc.VectorSubcoreMesh(core_axis_name="c", subcore_axis_name="s")

    @functools.partial(
        pl.kernel,
        out_type=jax.ShapeDtypeStruct((2 * N,), jnp.float32),
        mesh=mesh,
        compiler_params=pltpu.CompilerParams(
            needs_layout_passes=False, use_tc_tiling_on_sc=False),
        scratch_types=dict(
            deg_sp=pltpu.VMEM_SHARED((N,), jnp.float32),
            col_res=pltpu.VMEM((_DCH, _EK), jnp.int32),
            ew_res=pltpu.VMEM((_DCH, _EK), jnp.float32),
            buf_v=pltpu.VMEM((_EK,), jnp.float32),
        ),
    )
    def run(col_h, ew_h, out_h, *, deg_sp, col_res, ew_res, buf_v):
        c = lax.axis_index("c")
        s = lax.axis_index("s")

        pltpu.sync_copy(col_h.at[c, s], col_res)
        pltpu.sync_copy(ew_h.at[c, s], ew_res)

        for q in range(_EK // 16):
            buf_v[pl.ds(q * 16, 16)] = jnp.zeros((16,), jnp.float32)

        n_node_chunks = N // _EK
        n_iter = (n_node_chunks + _TILES - 1) // _TILES

        def zero_body(m, _):
            j = s + m * _TILES

            @pl.when(j < n_node_chunks)
            def _():
                off = pl.multiple_of(j * _EK, 8)
                pltpu.sync_copy(buf_v, deg_sp.at[pl.ds(off, _EK)])
            return _
        lax.fori_loop(0, n_iter, zero_body, None, unroll=False)

        plsc.subcore_barrier()

        def edge_body(j, _):
            pltpu.sync_copy(ew_res.at[j], deg_sp.at[col_res.at[j]], add=True)
            return _
        lax.fori_loop(0, _DCH, edge_body, None, unroll=False)

        plsc.subcore_barrier()

        def out_body(m, _):
            j = s + m * _TILES

            @pl.when(j < n_node_chunks)
            def _():
                off = pl.multiple_of(j * _EK, 8)
                pltpu.sync_copy(deg_sp.at[pl.ds(off, _EK)], buf_v)
                out_off = pl.multiple_of(c * N + off, 8)
                pltpu.sync_copy(buf_v, out_h.at[pl.ds(out_off, _EK)])
            return _
        lax.fori_loop(0, n_iter, out_body, None, unroll=False)

    return run(col4, ew4)


_NCH = E // _TILES // _EK
_SUP = 50
_NBUF = 5
_NST = 80


def _spmm_sc(row3, col3, ew3, gA, gB):
    mesh = plsc.VectorSubcoreMesh(core_axis_name="c", subcore_axis_name="s")

    @functools.partial(
        pl.kernel,
        out_type=[jax.ShapeDtypeStruct((N, DH), jnp.float32),
                  jax.ShapeDtypeStruct((N, DH), jnp.float32)],
        mesh=mesh,
        compiler_params=pltpu.CompilerParams(
            needs_layout_passes=False, use_tc_tiling_on_sc=False),
        scratch_types=dict(
            g_sp=pltpu.VMEM_SHARED((N, DH), jnp.float32),
            acc_sp=pltpu.VMEM_SHARED((N, DH), jnp.float32),
            ridx_res=pltpu.VMEM((_SUP, _EK), jnp.int32),
            cidx_res=pltpu.VMEM((_SUP, _EK), jnp.int32),
            ew_res=pltpu.VMEM((_SUP, _EK + 16), jnp.float32),
            rows_bufs=[pltpu.VMEM((_EK, DH), jnp.float32)
                       for _ in range(_NBUF)],
            stage_v=pltpu.VMEM((_NST, DH), jnp.float32),
            g_sems=[pltpu.SemaphoreType.DMA for _ in range(_NBUF)],
            s_sems=[pltpu.SemaphoreType.DMA for _ in range(_NBUF)],
        ),
    )
    def run(row_h, col_h, ew_h, ga_h, gb_h, sa_h, sb_h, *,
            g_sp, acc_sp, ridx_res, cidx_res, ew_res, rows_bufs,
            stage_v, g_sems, s_sems):
        c = lax.axis_index("c")
        s = lax.axis_index("s")
        n_node_chunks = N // _NST
        n_iter = (n_node_chunks + _TILES - 1) // _TILES

        def stage_in(g_h):
            def body(m, _):
                j = s + m * _TILES

                @pl.when(j < n_node_chunks)
                def _():
                    sl = pl.ds(pl.multiple_of(j * _NST, 8), _NST)
                    pltpu.sync_copy(g_h.at[sl, :], stage_v)
                    pltpu.sync_copy(stage_v, g_sp.at[sl, :])
                    pltpu.sync_copy(stage_v, acc_sp.at[sl, :])
                return _
            lax.fori_loop(0, n_iter, body, None, unroll=False)

        @pl.when(c == 0)
        def _():
            stage_in(ga_h)

        @pl.when(c == 1)
        def _():
            stage_in(gb_h)

        plsc.subcore_barrier()

        def scale(rows_ref, j):
            @functools.partial(plsc.parallel_loop, 0, _EK, unroll=8)
            def _(e):
                bc = plsc.load_gather(
                    ew_res,
                    [jnp.full((16,), j, jnp.int32),
                     jnp.full((16,), e + 16, jnp.int32)])
                for q in range(DH // 16):
                    sl = pl.ds(q * 16, 16)
                    rows_ref[e, sl] = rows_ref[e, sl] * bc

        def gather_start(j, rows_ref, sem):
            return pltpu.async_copy(g_sp.at[ridx_res.at[j]], rows_ref, sem)

        def gather_wait(j, rows_ref, sem):
            pltpu.make_async_copy(
                g_sp.at[ridx_res.at[j]], rows_ref, sem).wait()

        def scatter_start(j, rows_ref, sem):
            return pltpu.async_copy(
                rows_ref, acc_sp.at[cidx_res.at[j]], sem, add=True)

        def scatter_wait(j, rows_ref, sem):
            pltpu.make_async_copy(
                rows_ref, acc_sp.at[cidx_res.at[j]], sem).wait()

        n_steps = _SUP // _NBUF
        last_t = n_steps - 1

        def super_body(u, _):
            pltpu.sync_copy(row_h.at[s, pl.ds(u * _SUP, _SUP)], ridx_res)
            pltpu.sync_copy(col_h.at[s, pl.ds(u * _SUP, _SUP)], cidx_res)
            pltpu.sync_copy(ew_h.at[s, pl.ds(u * _SUP, _SUP)],
                            ew_res.at[:, pl.ds(16, _EK)])
            for k in range(3):
                gather_start(k, rows_bufs[k], g_sems[k])

            def edge_body(t, _):
                for k in range(_NBUF):
                    j = t * _NBUF + k
                    buf = rows_bufs[k]
                    gather_wait(j, buf, g_sems[k])
                    scale(buf, j)
                    scatter_start(j, buf, s_sems[k])
                    nk = (k + 3) % _NBUF
                    if k >= 2:
                        scatter_wait(j - 2, rows_bufs[nk], s_sems[nk])

                        @pl.when(t < last_t)
                        def _():
                            gather_start(j + 3, rows_bufs[nk], g_sems[nk])
                    else:
                        @pl.when(t > 0)
                        def _():
                            scatter_wait(j - 2, rows_bufs[nk], s_sems[nk])
                        gather_start(j + 3, rows_bufs[nk], g_sems[nk])
                return _
            lax.fori_loop(0, n_steps, edge_body, None, unroll=False)
            scatter_wait(_SUP - 2, rows_bufs[(_SUP - 2) % _NBUF],
                         s_sems[(_SUP - 2) % _NBUF])
            scatter_wait(_SUP - 1, rows_bufs[(_SUP - 1) % _NBUF],
                         s_sems[(_SUP - 1) % _NBUF])
            return _
        lax.fori_loop(0, _NCH // _SUP, super_body, None, unroll=False)

        plsc.subcore_barrier()

        def stage_out(s_h):
            def body(m, _):
                j = s + m * _TILES

                @pl.when(j < n_node_chunks)
                def _():
                    sl = pl.ds(pl.multiple_of(j * _NST, 8), _NST)
                    pltpu.sync_copy(acc_sp.at[sl, :], stage_v)
                    pltpu.sync_copy(stage_v, s_h.at[sl, :])
                return _
            lax.fori_loop(0, n_iter, body, None, unroll=False)

        @pl.when(c == 0)
        def _():
            stage_out(sa_h)

        @pl.when(c == 1)
        def _():
            stage_out(sb_h)

    return run(row3, col3, ew3, gA, gB)


def _dg_body(p0_ref, p1_ref, x_ref, wa_ref, wb_ref, d_ref, ga_ref, gb_ref):
    d = jax.lax.rsqrt(p0_ref[...] + p1_ref[...] + 1.0)
    x = x_ref[...]
    ga_ref[...] = d * jnp.dot(x, wa_ref[...], preferred_element_type=jnp.float32)
    gb_ref[...] = d * jnp.dot(x, wb_ref[...], preferred_element_type=jnp.float32)
    d_ref[...] = d


def _d_and_g1(deg_parts, node_feats, W1):
    BN = 1000
    return pl.pallas_call(
        _dg_body,
        grid=(N // BN,),
        in_specs=[
            pl.BlockSpec((BN, 1), lambda i: (i, 0)),
            pl.BlockSpec((BN, 1), lambda i: (i, 0)),
            pl.BlockSpec((BN, D), lambda i: (i, 0)),
            pl.BlockSpec((D, DH), lambda i: (0, 0)),
            pl.BlockSpec((D, DH), lambda i: (0, 0)),
        ],
        out_specs=[
            pl.BlockSpec((BN, 1), lambda i: (i, 0)),
            pl.BlockSpec((BN, DH), lambda i: (i, 0)),
            pl.BlockSpec((BN, DH), lambda i: (i, 0)),
        ],
        out_shape=[
            jax.ShapeDtypeStruct((N, 1), jnp.float32),
            jax.ShapeDtypeStruct((N, DH), jnp.float32),
            jax.ShapeDtypeStruct((N, DH), jnp.float32),
        ],
    )(deg_parts[:N].reshape(N, 1), deg_parts[N:].reshape(N, 1),
      node_feats, W1[:, :DH], W1[:, DH:])


def _mid_body(sa_ref, sb_ref, d_ref, b1_ref, wa_ref, wb_ref, oa_ref, ob_ref):
    d = d_ref[...]
    b1 = b1_ref[...]
    xa = jnp.tanh(d * sa_ref[...] + b1[:, :DH])
    xb = jnp.tanh(d * sb_ref[...] + b1[:, DH:])
    x1 = jnp.concatenate([xa, xb], axis=1)
    oa_ref[...] = d * jnp.dot(x1, wa_ref[...], preferred_element_type=jnp.float32)
    ob_ref[...] = d * jnp.dot(x1, wb_ref[...], preferred_element_type=jnp.float32)


def _mid(SA, SB, dcol, b1, W2):
    BN = 1000
    return pl.pallas_call(
        _mid_body,
        grid=(N // BN,),
        in_specs=[
            pl.BlockSpec((BN, DH), lambda i: (i, 0)),
            pl.BlockSpec((BN, DH), lambda i: (i, 0)),
            pl.BlockSpec((BN, 1), lambda i: (i, 0)),
            pl.BlockSpec((1, D), lambda i: (0, 0)),
            pl.BlockSpec((D, DH), lambda i: (0, 0)),
            pl.BlockSpec((D, DH), lambda i: (0, 0)),
        ],
        out_specs=[
            pl.BlockSpec((BN, DH), lambda i: (i, 0)),
            pl.BlockSpec((BN, DH), lambda i: (i, 0)),
        ],
        out_shape=[
            jax.ShapeDtypeStruct((N, DH), jnp.float32),
            jax.ShapeDtypeStruct((N, DH), jnp.float32),
        ],
    )(SA, SB, dcol, b1.reshape(1, D), W2[:, :DH], W2[:, DH:])


def _fin_body(sa_ref, sb_ref, d_ref, b2_ref, x_ref, o_ref):
    d = d_ref[...]
    ya = d * sa_ref[...]
    yb = d * sb_ref[...]
    o_ref[...] = jnp.concatenate([ya, yb], axis=1) + b2_ref[...] + x_ref[...]


def _final(SA, SB, dcol, b2, node_feats):
    BN = 1000
    return pl.pallas_call(
        _fin_body,
        grid=(N // BN,),
        in_specs=[
            pl.BlockSpec((BN, DH), lambda i: (i, 0)),
            pl.BlockSpec((BN, DH), lambda i: (i, 0)),
            pl.BlockSpec((BN, 1), lambda i: (i, 0)),
            pl.BlockSpec((1, D), lambda i: (0, 0)),
            pl.BlockSpec((BN, D), lambda i: (i, 0)),
        ],
        out_specs=pl.BlockSpec((BN, D), lambda i: (i, 0)),
        out_shape=jax.ShapeDtypeStruct((N, D), jnp.float32),
    )(SA, SB, dcol, b2.reshape(1, D), node_feats)


def kernel(edge_index, edge_feats, node_feats, Wet, bet, W1, b1, W2, b2):
    row = edge_index[0]
    col = edge_index[1]
    ew = _edge_weights(edge_feats, Wet, bet)
    deg_parts = _degree_sc(col.reshape(2, _TILES, _DCH, _EK),
                           ew.reshape(2, _TILES, _DCH, _EK))
    dcol, g1A, g1B = _d_and_g1(deg_parts, node_feats, W1)
    row3 = row.reshape(_TILES, _NCH, _EK)
    col3 = col.reshape(_TILES, _NCH, _EK)
    ew3 = ew.reshape(_TILES, _NCH, _EK)
    S1A, S1B = _spmm_sc(row3, col3, ew3, g1A, g1B)
    g2A, g2B = _mid(S1A, S1B, dcol, b1, W2)
    S2A, S2B = _spmm_sc(row3, col3, ew3, g2A, g2B)
    return _final(S2A, S2B, dcol, b2, node_feats)
